# baseline (speedup 1.0000x reference)
import functools

import jax
import jax.numpy as jnp
from jax import lax
from jax.experimental import pallas as pl
from jax.experimental.pallas import tpu as pltpu
from jax.experimental.pallas import tpu_sc as plsc

BATCH = 2
POSN = 2048
D_MODEL = 768
NC, NS = 2, 16
NW = NC * NS
B_PER_W = BATCH * POSN // NW
W_PER_ROW = POSN // B_PER_W
CHUNK = 32
NCHUNK = B_PER_W // CHUNK
NBUF = 4

_mesh = plsc.VectorSubcoreMesh(core_axis_name="c", subcore_axis_name="s")


@functools.partial(
    pl.kernel,
    mesh=_mesh,
    out_type=jax.ShapeDtypeStruct((BATCH, POSN, D_MODEL), jnp.float32),
    scratch_types=[
        pltpu.VMEM((B_PER_W,), jnp.int32),
        pltpu.VMEM((NBUF, CHUNK, D_MODEL), jnp.float32),
        pltpu.SemaphoreType.DMA((NBUF,)),
        pltpu.SemaphoreType.DMA((NBUF,)),
    ],
)
def _embed_sc(tokens_hbm, table_hbm, out_hbm, idx_v, rows_v, gsem, osem):
    wid = lax.axis_index("s") * NC + lax.axis_index("c")
    row = wid // W_PER_ROW
    col = (wid % W_PER_ROW) * B_PER_W
    pltpu.sync_copy(tokens_hbm.at[row, pl.ds(col, B_PER_W)], idx_v)

    def gather(c, b):
        return pltpu.async_copy(
            table_hbm.at[idx_v.at[pl.ds(c * CHUNK, CHUNK)]],
            rows_v.at[b],
            gsem.at[b],
        )

    def put(c, b):
        return pltpu.async_copy(
            rows_v.at[b],
            out_hbm.at[row, pl.ds(col + c * CHUNK, CHUNK)],
            osem.at[b],
        )

    g = [None] * NCHUNK
    o = [None] * NCHUNK
    for c in range(NCHUNK):
        if c >= NBUF:
            o[c - NBUF].wait()
        g[c] = gather(c, c % NBUF)
        if c >= NBUF - 1:
            j = c - (NBUF - 1)
            g[j].wait()
            o[j] = put(j, j % NBUF)
    for j in range(NCHUNK - NBUF + 1, NCHUNK):
        g[j].wait()
        o[j] = put(j, j % NBUF)
    for j in range(NCHUNK - NBUF, NCHUNK):
        o[j].wait()


@jax.jit
def kernel(tokens, W_E):
    lo = _embed_sc(tokens[:BATCH], W_E)
    hi = _embed_sc(tokens[BATCH:], W_E)
    return jnp.concatenate([lo, hi], axis=0)


# speedup vs baseline: 1.6220x; 1.6220x over previous
import functools

import jax
import jax.numpy as jnp
from jax import lax
from jax.experimental import pallas as pl
from jax.experimental.pallas import tpu as pltpu
from jax.experimental.pallas import tpu_sc as plsc

BATCH = 4
POSN = 2048
D_MODEL = 768
NC, NS = 2, 16
NW = NC * NS
B_PER_W = BATCH * POSN // NW
W_PER_ROW = POSN // B_PER_W
CHUNK = 32
NCHUNK = B_PER_W // CHUNK
NBUF = 5

_mesh = plsc.VectorSubcoreMesh(core_axis_name="c", subcore_axis_name="s")


@functools.partial(
    pl.kernel,
    mesh=_mesh,
    out_type=jax.ShapeDtypeStruct((BATCH, POSN, D_MODEL), jnp.float32),
    scratch_types=[
        pltpu.VMEM((B_PER_W,), jnp.int32),
        pltpu.VMEM((NBUF, CHUNK, D_MODEL), jnp.float32),
        pltpu.SemaphoreType.DMA((NBUF,)),
        pltpu.SemaphoreType.DMA((NBUF,)),
    ],
)
def _embed_sc(tokens_hbm, table_hbm, out_hbm, idx_v, rows_v, gsem, osem):
    wid = lax.axis_index("s") * NC + lax.axis_index("c")
    row = wid // W_PER_ROW
    col = (wid % W_PER_ROW) * B_PER_W
    pltpu.sync_copy(tokens_hbm.at[row, pl.ds(col, B_PER_W)], idx_v)

    def gather(c, b):
        return pltpu.async_copy(
            table_hbm.at[idx_v.at[pl.ds(c * CHUNK, CHUNK)]],
            rows_v.at[b],
            gsem.at[b],
        )

    def put(c, b):
        return pltpu.async_copy(
            rows_v.at[b],
            out_hbm.at[row, pl.ds(col + c * CHUNK, CHUNK)],
            osem.at[b],
        )

    g = [None] * NCHUNK
    o = [None] * NCHUNK
    for c in range(NCHUNK):
        if c >= NBUF:
            o[c - NBUF].wait()
        g[c] = gather(c, c % NBUF)
        if c >= NBUF - 1:
            j = c - (NBUF - 1)
            g[j].wait()
            o[j] = put(j, j % NBUF)
    for j in range(NCHUNK - NBUF + 1, NCHUNK):
        g[j].wait()
        o[j] = put(j, j % NBUF)
    for j in range(NCHUNK - NBUF, NCHUNK):
        o[j].wait()


@jax.jit
def kernel(tokens, W_E):
    return _embed_sc(tokens, W_E)
